# baseline (speedup 1.0000x reference)
import functools

import jax
import jax.numpy as jnp
from jax import lax
from jax.experimental import pallas as pl
from jax.experimental.pallas import tpu as pltpu
from jax.experimental.pallas import tpu_sc as plsc

_NUM_CORES = 2
_NUM_SUBCORES = 16
_NUM_WORKERS = _NUM_CORES * _NUM_SUBCORES
_L = 16


def kernel(inputs, token_table, pos_table):
    B, S = inputs.shape
    V, D = token_table.shape
    idx = inputs.reshape(-1).astype(jnp.int32)
    tok2 = token_table.reshape(V // 2, 2 * D)
    seq_per_w = B // _NUM_WORKERS

    mesh = plsc.VectorSubcoreMesh(core_axis_name="c", subcore_axis_name="s")
    NV = S // _L + (1 if S % _L else 0)

    @functools.partial(
        pl.kernel,
        mesh=mesh,
        out_type=jax.ShapeDtypeStruct((B * S // 2, 2 * D), jnp.float32),
        scratch_types=[
            [pltpu.VMEM((S,), jnp.int32) for _ in range(2)],
            [pltpu.VMEM((S,), jnp.int32) for _ in range(2)],
            [pltpu.VMEM((S + 24,), jnp.int32) for _ in range(2)],
            [pltpu.VMEM((S, 2 * D), jnp.float32) for _ in range(2)],
            [pltpu.VMEM((S, 2 * D), jnp.float32) for _ in range(2)],
            pltpu.VMEM((S, D), jnp.float32),
            pltpu.SemaphoreType.DMA((2,)),
            pltpu.SemaphoreType.DMA((2,)),
            pltpu.SemaphoreType.DMA((2,)),
        ],
        compiler_params=pltpu.CompilerParams(needs_layout_passes=False),
    )
    def sc_kernel(idx_hbm, tok_hbm, pos_hbm, out_hbm, idxs, pidx, h64, gath,
                  outb, pos_v, gsem, ssem, isem):
        wid = lax.axis_index("s") * _NUM_CORES + lax.axis_index("c")
        base = wid * seq_per_w * S
        pltpu.sync_copy(pos_hbm, pos_v)
        iota = lax.iota(jnp.int32, _L)

        def idx_start(s):
            b = s % 2
            return pltpu.async_copy(idx_hbm.at[pl.ds(base + s * S, S)],
                                    idxs[b], isem.at[b])

        def prep(s):
            b = s % 2
            for i in range(NV):
                o = min(i * _L, S - _L)
                v = idxs[b][pl.ds(o, _L)]
                pidx[b][pl.ds(o, _L)] = lax.shift_right_logical(v, 1)
                h64[b][pl.ds(o, _L)] = lax.shift_left(
                    lax.bitwise_and(v, 1), 6)

        def gather_start(s):
            b = s % 2
            return pltpu.async_copy(tok_hbm.at[pidx[b]], gath[b], gsem.at[b])

        def extract(s):
            b = s % 2
            ob = (s // 2) % 2
            row0 = (s % 2) * (S // 2)

            def per_pair(q, _):
                h2 = h64[b][pl.ds(2 * q, _L)]
                for par in range(2):
                    r = 2 * q + par
                    lane0 = h2[par]
                    for c in range(D // _L):
                        lanes = lane0 + (c * _L + iota)
                        rows = jnp.full((_L,), r, dtype=jnp.int32)
                        vals = plsc.load_gather(gath[b], [rows, lanes])
                        sl = pl.ds(c * _L, _L)
                        outb[ob][row0 + q, pl.ds(par * D + c * _L, _L)] = (
                            vals + pos_v[r, sl])
                return ()

            lax.fori_loop(0, S // 2, per_pair, ())

        def store_start(g):
            ob = g % 2
            pair_base = (wid * seq_per_w + 2 * g) * (S // 2)
            return pltpu.async_copy(outb[ob], out_hbm.at[pl.ds(pair_base, S)],
                                    ssem.at[ob])

        st = [None] * (seq_per_w // 2)
        iv = [idx_start(0), idx_start(1)]
        iv[0].wait()
        prep(0)
        gs = [gather_start(0), None]
        iv[0] = idx_start(2)
        iv[1].wait()
        prep(1)
        gs[1] = gather_start(1)
        iv[1] = idx_start(3)
        for s in range(seq_per_w):
            g = s // 2
            if s >= 4 and s % 2 == 0:
                st[g - 2].wait()
            gs[s % 2].wait()
            extract(s)
            if s + 2 < seq_per_w:
                iv[s % 2].wait()
                prep(s + 2)
                gs[s % 2] = gather_start(s + 2)
                if s + 4 < seq_per_w:
                    iv[s % 2] = idx_start(s + 4)
            if s % 2 == 1:
                st[g] = store_start(g)
        st[seq_per_w // 2 - 1].wait()
        st[seq_per_w // 2 - 2].wait()

    out2 = sc_kernel(idx, tok2, pos_table)
    return out2.reshape(B, S, D)


# speedup vs baseline: 1.0766x; 1.0766x over previous
import functools

import jax
import jax.numpy as jnp
from jax import lax
from jax.experimental import pallas as pl
from jax.experimental.pallas import tpu as pltpu
from jax.experimental.pallas import tpu_sc as plsc

_NUM_CORES = 2
_NUM_SUBCORES = 16
_NUM_WORKERS = _NUM_CORES * _NUM_SUBCORES
_L = 16
_CH = 2048


def _convert_table(tok_t):
    D, V = tok_t.shape

    def body(x_ref, o_ref):
        xt = x_ref[...].T
        x3 = xt.reshape(_CH // 2, 2, D)
        o_ref[...] = jnp.concatenate([x3[:, 0, :], x3[:, 1, :]], axis=1)

    return pl.pallas_call(
        body,
        grid=(pl.cdiv(V, _CH),),
        in_specs=[pl.BlockSpec((D, _CH), lambda i: (0, i))],
        out_specs=pl.BlockSpec((_CH // 2, 2 * D), lambda i: (i, 0)),
        out_shape=jax.ShapeDtypeStruct((V // 2, 2 * D), jnp.float32),
    )(tok_t)


def kernel(inputs, token_table, pos_table):
    B, S = inputs.shape
    V, D = token_table.shape
    idx = inputs.reshape(-1).astype(jnp.int32)
    tok2 = _convert_table(token_table.T)
    pos2 = pos_table.reshape(S // 2, 2 * D)
    seq_per_w = B // _NUM_WORKERS

    mesh = plsc.VectorSubcoreMesh(core_axis_name="c", subcore_axis_name="s")
    NV = S // _L + (1 if S % _L else 0)

    @functools.partial(
        pl.kernel,
        mesh=mesh,
        out_type=jax.ShapeDtypeStruct((B * S // 2, 2 * D), jnp.float32),
        scratch_types=[
            [pltpu.VMEM((S,), jnp.int32) for _ in range(2)],
            [pltpu.VMEM((S,), jnp.int32) for _ in range(2)],
            [pltpu.VMEM((S + 24,), jnp.int32) for _ in range(2)],
            [pltpu.VMEM((S, 2 * D), jnp.float32) for _ in range(2)],
            [pltpu.VMEM((S, 2 * D), jnp.float32) for _ in range(2)],
            pltpu.VMEM((S // 2, 2 * D), jnp.float32),
            pltpu.SemaphoreType.DMA((2,)),
            pltpu.SemaphoreType.DMA((2,)),
            pltpu.SemaphoreType.DMA((2,)),
        ],
        compiler_params=pltpu.CompilerParams(needs_layout_passes=False),
    )
    def sc_kernel(idx_hbm, tok_hbm, pos_hbm, out_hbm, idxs, pidx, hv, gath,
                  outb, pos_v, gsem, ssem, isem):
        wid = lax.axis_index("s") * _NUM_CORES + lax.axis_index("c")
        base = wid * seq_per_w * S
        pltpu.sync_copy(pos_hbm, pos_v)

        def idx_start(s):
            b = s % 2
            return pltpu.async_copy(idx_hbm.at[pl.ds(base + s * S, S)],
                                    idxs[b], isem.at[b])

        def prep(s):
            b = s % 2
            for i in range(NV):
                o = min(i * _L, S - _L)
                v = idxs[b][pl.ds(o, _L)]
                pidx[b][pl.ds(o, _L)] = lax.shift_right_logical(v, 1)
                hv[b][pl.ds(o, _L)] = lax.bitwise_and(v, 1)

        def gather_start(s):
            b = s % 2
            return pltpu.async_copy(tok_hbm.at[pidx[b]], gath[b], gsem.at[b])

        def extract(s):
            b = s % 2
            ob = (s // 2) % 2
            row0 = (s % 2) * (S // 2)

            def per_pair(q, _):
                h2 = hv[b][pl.ds(2 * q, _L)]
                for par in range(2):
                    r = 2 * q + par
                    m = jnp.full((_L,), h2[par], dtype=jnp.int32) != 0
                    for c in range(D // _L):
                        lo = gath[b][r, pl.ds(c * _L, _L)]
                        hi = gath[b][r, pl.ds(D + c * _L, _L)]
                        sl = pl.ds(par * D + c * _L, _L)
                        outb[ob][row0 + q, sl] = (
                            jnp.where(m, hi, lo) + pos_v[q, sl])
                return ()

            lax.fori_loop(0, S // 2, per_pair, ())

        def store_start(g):
            ob = g % 2
            pair_base = (wid * seq_per_w + 2 * g) * (S // 2)
            return pltpu.async_copy(outb[ob], out_hbm.at[pl.ds(pair_base, S)],
                                    ssem.at[ob])

        st = [None] * (seq_per_w // 2)
        iv = [idx_start(0), idx_start(1)]
        iv[0].wait()
        prep(0)
        gs = [gather_start(0), None]
        iv[0] = idx_start(2)
        iv[1].wait()
        prep(1)
        gs[1] = gather_start(1)
        iv[1] = idx_start(3)
        for s in range(seq_per_w):
            g = s // 2
            if s >= 4 and s % 2 == 0:
                st[g - 2].wait()
            gs[s % 2].wait()
            extract(s)
            if s + 2 < seq_per_w:
                iv[s % 2].wait()
                prep(s + 2)
                gs[s % 2] = gather_start(s + 2)
                if s + 4 < seq_per_w:
                    iv[s % 2] = idx_start(s + 4)
            if s % 2 == 1:
                st[g] = store_start(g)
        st[seq_per_w // 2 - 1].wait()
        st[seq_per_w // 2 - 2].wait()

    out2 = sc_kernel(idx, tok2, pos2)
    return out2.reshape(B, S, D)


# speedup vs baseline: 1.2103x; 1.1242x over previous
import functools

import jax
import jax.numpy as jnp
from jax import lax
from jax.experimental import pallas as pl
from jax.experimental.pallas import tpu as pltpu
from jax.experimental.pallas import tpu_sc as plsc

_NUM_CORES = 2
_NUM_SUBCORES = 16
_NUM_WORKERS = _NUM_CORES * _NUM_SUBCORES
_L = 16
_CH = 2048


def _convert_table(tok_t):
    D, V = tok_t.shape

    H = _CH // 2
    n_blk = pl.cdiv(V, _CH)

    def body(x_ref, o_ref):
        o_ref[:, 0:D] = x_ref[:, 0:H].T
        o_ref[:, D:2 * D] = x_ref[:, H:_CH].T

    return pl.pallas_call(
        body,
        grid=(n_blk,),
        in_specs=[pl.BlockSpec((D, _CH), lambda i: (0, i))],
        out_specs=pl.BlockSpec((H, 2 * D), lambda i: (i, 0)),
        out_shape=jax.ShapeDtypeStruct((n_blk * H, 2 * D), jnp.float32),
    )(tok_t)


def kernel(inputs, token_table, pos_table):
    B, S = inputs.shape
    V, D = token_table.shape
    idx = inputs.reshape(-1).astype(jnp.int32)
    tok2 = _convert_table(token_table.T)
    pos2 = pos_table.reshape(S // 2, 2 * D)
    seq_per_w = B // _NUM_WORKERS

    mesh = plsc.VectorSubcoreMesh(core_axis_name="c", subcore_axis_name="s")
    NV = S // _L + (1 if S % _L else 0)

    @functools.partial(
        pl.kernel,
        mesh=mesh,
        out_type=jax.ShapeDtypeStruct((B * S // 2, 2 * D), jnp.float32),
        scratch_types=[
            [pltpu.VMEM((S,), jnp.int32) for _ in range(2)],
            [pltpu.VMEM((S,), jnp.int32) for _ in range(2)],
            [pltpu.VMEM((S + 24,), jnp.int32) for _ in range(2)],
            [pltpu.VMEM((S, 2 * D), jnp.float32) for _ in range(2)],
            [pltpu.VMEM((S, 2 * D), jnp.float32) for _ in range(2)],
            pltpu.VMEM((S // 2, 2 * D), jnp.float32),
            pltpu.SemaphoreType.DMA((2,)),
            pltpu.SemaphoreType.DMA((2,)),
            pltpu.SemaphoreType.DMA((2,)),
        ],
        compiler_params=pltpu.CompilerParams(needs_layout_passes=False),
    )
    def sc_kernel(idx_hbm, tok_hbm, pos_hbm, out_hbm, idxs, pidx, hv, gath,
                  outb, pos_v, gsem, ssem, isem):
        wid = lax.axis_index("s") * _NUM_CORES + lax.axis_index("c")
        base = wid * seq_per_w * S
        pltpu.sync_copy(pos_hbm, pos_v)

        def idx_start(s):
            b = s % 2
            return pltpu.async_copy(idx_hbm.at[pl.ds(base + s * S, S)],
                                    idxs[b], isem.at[b])

        def prep(s):
            b = s % 2
            for i in range(NV):
                o = min(i * _L, S - _L)
                v = idxs[b][pl.ds(o, _L)]
                pidx[b][pl.ds(o, _L)] = lax.bitwise_or(
                    lax.shift_left(lax.shift_right_logical(v, 11), 10),
                    lax.bitwise_and(v, 1023))
                hv[b][pl.ds(o, _L)] = lax.bitwise_and(
                    lax.shift_right_logical(v, 10), 1)

        def gather_start(s):
            b = s % 2
            return pltpu.async_copy(tok_hbm.at[pidx[b]], gath[b], gsem.at[b])

        def extract(s):
            b = s % 2
            ob = (s // 2) % 2
            row0 = (s % 2) * (S // 2)

            def per_pair(q, _):
                h2 = hv[b][pl.ds(2 * q, _L)]
                for par in range(2):
                    r = 2 * q + par
                    m = jnp.full((_L,), h2[par], dtype=jnp.int32) != 0
                    for c in range(D // _L):
                        lo = gath[b][r, pl.ds(c * _L, _L)]
                        hi = gath[b][r, pl.ds(D + c * _L, _L)]
                        sl = pl.ds(par * D + c * _L, _L)
                        outb[ob][row0 + q, sl] = (
                            jnp.where(m, hi, lo) + pos_v[q, sl])
                return ()

            lax.fori_loop(0, S // 2, per_pair, ())

        def store_start(g):
            ob = g % 2
            pair_base = (wid * seq_per_w + 2 * g) * (S // 2)
            return pltpu.async_copy(outb[ob], out_hbm.at[pl.ds(pair_base, S)],
                                    ssem.at[ob])

        st = [None] * (seq_per_w // 2)
        iv = [idx_start(0), idx_start(1)]
        iv[0].wait()
        prep(0)
        gs = [gather_start(0), None]
        iv[0] = idx_start(2)
        iv[1].wait()
        prep(1)
        gs[1] = gather_start(1)
        iv[1] = idx_start(3)
        for s in range(seq_per_w):
            g = s // 2
            if s >= 4 and s % 2 == 0:
                st[g - 2].wait()
            gs[s % 2].wait()
            extract(s)
            if s + 2 < seq_per_w:
                iv[s % 2].wait()
                prep(s + 2)
                gs[s % 2] = gather_start(s + 2)
                if s + 4 < seq_per_w:
                    iv[s % 2] = idx_start(s + 4)
            if s % 2 == 1:
                st[g] = store_start(g)
        st[seq_per_w // 2 - 1].wait()
        st[seq_per_w // 2 - 2].wait()

    out2 = sc_kernel(idx, tok2, pos2)
    return out2.reshape(B, S, D)


# speedup vs baseline: 1.6082x; 1.3288x over previous
import functools

import jax
import jax.numpy as jnp
from jax import lax
from jax.experimental import pallas as pl
from jax.experimental.pallas import tpu as pltpu
from jax.experimental.pallas import tpu_sc as plsc

_NUM_CORES = 2
_NUM_SUBCORES = 16
_NUM_WORKERS = _NUM_CORES * _NUM_SUBCORES
_L = 16
_CH = 8192
_HB = _CH // 2


def _convert_table(tok_t):
    D, V = tok_t.shape

    H = _CH // 2
    n_blk = pl.cdiv(V, _CH)

    def body(x_ref, o_ref):
        o_ref[:, 0:D] = x_ref[:, 0:H].T
        o_ref[:, D:2 * D] = x_ref[:, H:_CH].T

    return pl.pallas_call(
        body,
        grid=(n_blk,),
        in_specs=[pl.BlockSpec((D, _CH), lambda i: (0, i))],
        out_specs=pl.BlockSpec((H, 2 * D), lambda i: (i, 0)),
        out_shape=jax.ShapeDtypeStruct((n_blk * H, 2 * D), jnp.float32),
    )(tok_t)


def kernel(inputs, token_table, pos_table):
    B, S = inputs.shape
    V, D = token_table.shape
    idx = inputs.reshape(-1).astype(jnp.int32)
    tok2 = _convert_table(token_table.T)
    pos2 = pos_table.reshape(S // 2, 2 * D)
    seq_per_w = B // _NUM_WORKERS

    mesh = plsc.VectorSubcoreMesh(core_axis_name="c", subcore_axis_name="s")
    NV = S // _L + (1 if S % _L else 0)

    @functools.partial(
        pl.kernel,
        mesh=mesh,
        out_type=jax.ShapeDtypeStruct((B * S // 2, 2 * D), jnp.float32),
        scratch_types=[
            [pltpu.VMEM((S,), jnp.int32) for _ in range(2)],
            [pltpu.VMEM((S,), jnp.int32) for _ in range(2)],
            [pltpu.VMEM((S + 24,), jnp.int32) for _ in range(2)],
            [pltpu.VMEM((S, 2 * D), jnp.float32) for _ in range(2)],
            [pltpu.VMEM((S, 2 * D), jnp.float32) for _ in range(2)],
            pltpu.VMEM((S // 2, 2 * D), jnp.float32),
            pltpu.SemaphoreType.DMA((2,)),
            pltpu.SemaphoreType.DMA((2,)),
            pltpu.SemaphoreType.DMA((2,)),
        ],
        compiler_params=pltpu.CompilerParams(needs_layout_passes=False),
    )
    def sc_kernel(idx_hbm, tok_hbm, pos_hbm, out_hbm, idxs, pidx, hv, gath,
                  outb, pos_v, gsem, ssem, isem):
        wid = lax.axis_index("s") * _NUM_CORES + lax.axis_index("c")
        base = wid * seq_per_w * S
        pltpu.sync_copy(pos_hbm, pos_v)

        def idx_start(s):
            b = s % 2
            return pltpu.async_copy(idx_hbm.at[pl.ds(base + s * S, S)],
                                    idxs[b], isem.at[b])

        def prep(s):
            b = s % 2
            for i in range(NV):
                o = min(i * _L, S - _L)
                v = idxs[b][pl.ds(o, _L)]
                pidx[b][pl.ds(o, _L)] = lax.bitwise_or(
                    lax.shift_left(lax.shift_right_logical(v, 13), 12),
                    lax.bitwise_and(v, _HB - 1))
                hv[b][pl.ds(o, _L)] = lax.bitwise_and(
                    lax.shift_right_logical(v, 12), 1)

        def gather_start(s):
            b = s % 2
            return pltpu.async_copy(tok_hbm.at[pidx[b]], gath[b], gsem.at[b])

        def extract(s):
            b = s % 2
            ob = (s // 2) % 2
            row0 = (s % 2) * (S // 2)

            def per_pair(q, _):
                h2 = hv[b][pl.ds(2 * q, _L)]
                for par in range(2):
                    r = 2 * q + par
                    m = jnp.full((_L,), h2[par], dtype=jnp.int32) != 0
                    for c in range(D // _L):
                        lo = gath[b][r, pl.ds(c * _L, _L)]
                        hi = gath[b][r, pl.ds(D + c * _L, _L)]
                        sl = pl.ds(par * D + c * _L, _L)
                        outb[ob][row0 + q, sl] = (
                            jnp.where(m, hi, lo) + pos_v[q, sl])
                return ()

            lax.fori_loop(0, S // 2, per_pair, ())

        def store_start(g):
            ob = g % 2
            pair_base = (wid * seq_per_w + 2 * g) * (S // 2)
            return pltpu.async_copy(outb[ob], out_hbm.at[pl.ds(pair_base, S)],
                                    ssem.at[ob])

        st = [None] * (seq_per_w // 2)
        iv = [idx_start(0), idx_start(1)]
        iv[0].wait()
        prep(0)
        gs = [gather_start(0), None]
        iv[0] = idx_start(2)
        iv[1].wait()
        prep(1)
        gs[1] = gather_start(1)
        iv[1] = idx_start(3)
        for s in range(seq_per_w):
            g = s // 2
            if s >= 4 and s % 2 == 0:
                st[g - 2].wait()
            gs[s % 2].wait()
            extract(s)
            if s + 2 < seq_per_w:
                iv[s % 2].wait()
                prep(s + 2)
                gs[s % 2] = gather_start(s + 2)
                if s + 4 < seq_per_w:
                    iv[s % 2] = idx_start(s + 4)
            if s % 2 == 1:
                st[g] = store_start(g)
        st[seq_per_w // 2 - 1].wait()
        st[seq_per_w // 2 - 2].wait()

    out2 = sc_kernel(idx, tok2, pos2)
    return out2.reshape(B, S, D)


# speedup vs baseline: 1.7008x; 1.0576x over previous
import functools

import jax
import jax.numpy as jnp
from jax import lax
from jax.experimental import pallas as pl
from jax.experimental.pallas import tpu as pltpu
from jax.experimental.pallas import tpu_sc as plsc

_NUM_CORES = 2
_NUM_SUBCORES = 16
_NUM_WORKERS = _NUM_CORES * _NUM_SUBCORES
_L = 16
_CH = 16384
_HB = _CH // 2
_SH = _CH.bit_length() - 1


def _convert_table(tok_t):
    D, V = tok_t.shape

    H = _CH // 2
    n_blk = pl.cdiv(V, _CH)

    def body(x_ref, o_ref):
        o_ref[:, 0:D] = x_ref[:, 0:H].T
        o_ref[:, D:2 * D] = x_ref[:, H:_CH].T

    return pl.pallas_call(
        body,
        grid=(n_blk,),
        in_specs=[pl.BlockSpec((D, _CH), lambda i: (0, i))],
        out_specs=pl.BlockSpec((H, 2 * D), lambda i: (i, 0)),
        out_shape=jax.ShapeDtypeStruct((n_blk * H, 2 * D), jnp.float32),
    )(tok_t)


def kernel(inputs, token_table, pos_table):
    B, S = inputs.shape
    V, D = token_table.shape
    idx = inputs.reshape(-1).astype(jnp.int32)
    tok2 = _convert_table(token_table.T)
    pos2 = pos_table.reshape(S // 2, 2 * D)
    seq_per_w = B // _NUM_WORKERS

    mesh = plsc.VectorSubcoreMesh(core_axis_name="c", subcore_axis_name="s")
    NV = S // _L + (1 if S % _L else 0)

    @functools.partial(
        pl.kernel,
        mesh=mesh,
        out_type=jax.ShapeDtypeStruct((B * S // 2, 2 * D), jnp.float32),
        scratch_types=[
            [pltpu.VMEM((S,), jnp.int32) for _ in range(2)],
            [pltpu.VMEM((S,), jnp.int32) for _ in range(2)],
            [pltpu.VMEM((S + 24,), jnp.int32) for _ in range(2)],
            [pltpu.VMEM((S, 2 * D), jnp.float32) for _ in range(2)],
            [pltpu.VMEM((S, 2 * D), jnp.float32) for _ in range(2)],
            pltpu.VMEM((S // 2, 2 * D), jnp.float32),
            pltpu.SemaphoreType.DMA((2,)),
            pltpu.SemaphoreType.DMA((2,)),
            pltpu.SemaphoreType.DMA((2,)),
        ],
        compiler_params=pltpu.CompilerParams(needs_layout_passes=False),
    )
    def sc_kernel(idx_hbm, tok_hbm, pos_hbm, out_hbm, idxs, pidx, hv, gath,
                  outb, pos_v, gsem, ssem, isem):
        wid = lax.axis_index("s") * _NUM_CORES + lax.axis_index("c")
        base = wid * seq_per_w * S
        pltpu.sync_copy(pos_hbm, pos_v)

        def idx_start(s):
            b = s % 2
            return pltpu.async_copy(idx_hbm.at[pl.ds(base + s * S, S)],
                                    idxs[b], isem.at[b])

        def prep(s):
            b = s % 2
            for i in range(NV):
                o = min(i * _L, S - _L)
                v = idxs[b][pl.ds(o, _L)]
                pidx[b][pl.ds(o, _L)] = lax.bitwise_or(
                    lax.shift_left(lax.shift_right_logical(v, _SH), _SH - 1),
                    lax.bitwise_and(v, _HB - 1))
                hv[b][pl.ds(o, _L)] = lax.bitwise_and(
                    lax.shift_right_logical(v, _SH - 1), 1)

        def gather_start(s):
            b = s % 2
            return pltpu.async_copy(tok_hbm.at[pidx[b]], gath[b], gsem.at[b])

        def extract(s):
            b = s % 2
            ob = (s // 2) % 2
            row0 = (s % 2) * (S // 2)

            def per_pair(q, _):
                h2 = hv[b][pl.ds(2 * q, _L)]
                for par in range(2):
                    r = 2 * q + par
                    m = jnp.full((_L,), h2[par], dtype=jnp.int32) != 0
                    for c in range(D // _L):
                        lo = gath[b][r, pl.ds(c * _L, _L)]
                        hi = gath[b][r, pl.ds(D + c * _L, _L)]
                        sl = pl.ds(par * D + c * _L, _L)
                        outb[ob][row0 + q, sl] = (
                            jnp.where(m, hi, lo) + pos_v[q, sl])
                return ()

            lax.fori_loop(0, S // 2, per_pair, ())

        def store_start(g):
            ob = g % 2
            pair_base = (wid * seq_per_w + 2 * g) * (S // 2)
            return pltpu.async_copy(outb[ob], out_hbm.at[pl.ds(pair_base, S)],
                                    ssem.at[ob])

        st = [None] * (seq_per_w // 2)
        iv = [idx_start(0), idx_start(1)]
        iv[0].wait()
        prep(0)
        gs = [gather_start(0), None]
        iv[0] = idx_start(2)
        iv[1].wait()
        prep(1)
        gs[1] = gather_start(1)
        iv[1] = idx_start(3)
        for s in range(seq_per_w):
            g = s // 2
            if s >= 4 and s % 2 == 0:
                st[g - 2].wait()
            gs[s % 2].wait()
            extract(s)
            if s + 2 < seq_per_w:
                iv[s % 2].wait()
                prep(s + 2)
                gs[s % 2] = gather_start(s + 2)
                if s + 4 < seq_per_w:
                    iv[s % 2] = idx_start(s + 4)
            if s % 2 == 1:
                st[g] = store_start(g)
        st[seq_per_w // 2 - 1].wait()
        st[seq_per_w // 2 - 2].wait()

    out2 = sc_kernel(idx, tok2, pos2)
    return out2.reshape(B, S, D)


# speedup vs baseline: 2.2936x; 1.3486x over previous
import functools

import jax
import jax.numpy as jnp
from jax import lax
from jax.experimental import pallas as pl
from jax.experimental.pallas import tpu as pltpu
from jax.experimental.pallas import tpu_sc as plsc

_NUM_CORES = 2
_NUM_SUBCORES = 16
_NUM_WORKERS = _NUM_CORES * _NUM_SUBCORES
_L = 16
_CH = 16384
_HB = _CH // 2
_SH = _CH.bit_length() - 1


def _convert_table(tok_t):
    D, V = tok_t.shape

    H = _CH // 2
    n_blk = pl.cdiv(V, _CH)

    def body(x_ref, o_ref):
        o_ref[:, 0:D] = x_ref[:, 0:H].T
        o_ref[:, D:2 * D] = x_ref[:, H:_CH].T

    return pl.pallas_call(
        body,
        grid=(n_blk,),
        in_specs=[pl.BlockSpec((D, _CH), lambda i: (0, i))],
        out_specs=pl.BlockSpec((H, 2 * D), lambda i: (i, 0)),
        out_shape=jax.ShapeDtypeStruct((n_blk * H, 2 * D), jnp.float32),
    )(tok_t)


def kernel(inputs, token_table, pos_table):
    B, S = inputs.shape
    V, D = token_table.shape
    idx = inputs.reshape(-1).astype(jnp.int32)
    tok2 = _convert_table(token_table.T)
    seq_per_w = B // _NUM_WORKERS

    mesh = plsc.VectorSubcoreMesh(core_axis_name="c", subcore_axis_name="s")
    NV = S // _L + (1 if S % _L else 0)

    @functools.partial(
        pl.kernel,
        mesh=mesh,
        out_type=jax.ShapeDtypeStruct((B, S, D), jnp.float32),
        scratch_types=[
            [pltpu.VMEM((S,), jnp.int32) for _ in range(2)],
            [pltpu.VMEM((S,), jnp.int32) for _ in range(2)],
            [pltpu.VMEM((S + 24,), jnp.int32) for _ in range(2)],
            [pltpu.VMEM((S, 2 * D), jnp.float32) for _ in range(2)],
            [pltpu.VMEM((S, D), jnp.float32) for _ in range(2)],
            pltpu.VMEM((S, D), jnp.float32),
            pltpu.SemaphoreType.DMA((2,)),
            pltpu.SemaphoreType.DMA((2,)),
            pltpu.SemaphoreType.DMA((2,)),
        ],
        compiler_params=pltpu.CompilerParams(needs_layout_passes=False),
    )
    def sc_kernel(idx_hbm, tok_hbm, pos_hbm, out_hbm, idxs, pidx, hv, gath,
                  outb, pos_v, gsem, ssem, isem):
        wid = lax.axis_index("s") * _NUM_CORES + lax.axis_index("c")
        base = wid * seq_per_w * S
        pltpu.sync_copy(pos_hbm, pos_v)

        def idx_start(s):
            b = s % 2
            return pltpu.async_copy(idx_hbm.at[pl.ds(base + s * S, S)],
                                    idxs[b], isem.at[b])

        def prep(s):
            b = s % 2
            for i in range(NV):
                o = min(i * _L, S - _L)
                v = idxs[b][pl.ds(o, _L)]
                pidx[b][pl.ds(o, _L)] = lax.bitwise_or(
                    lax.shift_left(lax.shift_right_logical(v, _SH), _SH - 1),
                    lax.bitwise_and(v, _HB - 1))
                hv[b][pl.ds(o, _L)] = lax.bitwise_and(
                    lax.shift_right_logical(v, _SH - 1), 1)

        def gather_start(s):
            b = s % 2
            return pltpu.async_copy(tok_hbm.at[pidx[b]], gath[b], gsem.at[b])

        def extract(s):
            b = s % 2

            def per_pair(q, _):
                h2 = hv[b][pl.ds(2 * q, _L)]
                for par in range(2):
                    r = 2 * q + par
                    m = jnp.full((_L,), h2[par], dtype=jnp.int32) != 0
                    for c in range(D // _L):
                        sl = pl.ds(c * _L, _L)
                        lo = gath[b][r, sl]
                        hi = gath[b][r, pl.ds(D + c * _L, _L)]
                        outb[b][r, sl] = jnp.where(m, hi, lo) + pos_v[r, sl]
                return ()

            lax.fori_loop(0, S // 2, per_pair, ())

        def store_start(s):
            return pltpu.async_copy(outb[s % 2],
                                    out_hbm.at[wid * seq_per_w + s],
                                    ssem.at[s % 2])

        st = [None] * seq_per_w
        iv = [idx_start(0), idx_start(1)]
        iv[0].wait()
        prep(0)
        gs = [gather_start(0), None]
        iv[0] = idx_start(2)
        iv[1].wait()
        prep(1)
        gs[1] = gather_start(1)
        iv[1] = idx_start(3)
        for s in range(seq_per_w):
            if s >= 2:
                st[s - 2].wait()
            gs[s % 2].wait()
            extract(s)
            if s + 2 < seq_per_w:
                iv[s % 2].wait()
                prep(s + 2)
                gs[s % 2] = gather_start(s + 2)
                if s + 4 < seq_per_w:
                    iv[s % 2] = idx_start(s + 4)
            st[s] = store_start(s)
        st[seq_per_w - 1].wait()
        st[seq_per_w - 2].wait()

    return sc_kernel(idx, tok2, pos_table)


# speedup vs baseline: 2.3813x; 1.0382x over previous
import functools

import jax
import jax.numpy as jnp
from jax import lax
from jax.experimental import pallas as pl
from jax.experimental.pallas import tpu as pltpu
from jax.experimental.pallas import tpu_sc as plsc

_NUM_CORES = 2
_NUM_SUBCORES = 16
_NUM_WORKERS = _NUM_CORES * _NUM_SUBCORES
_L = 16
_CH = 32768
_HB = _CH // 2
_SH = _CH.bit_length() - 1


def _convert_table(tok_t):
    D, V = tok_t.shape

    H = _CH // 2
    n_blk = pl.cdiv(V, _CH)

    def body(x_ref, o_ref):
        o_ref[:, 0:D] = x_ref[:, 0:H].T
        o_ref[:, D:2 * D] = x_ref[:, H:_CH].T

    return pl.pallas_call(
        body,
        grid=(n_blk,),
        in_specs=[pl.BlockSpec((D, _CH), lambda i: (0, i))],
        out_specs=pl.BlockSpec((H, 2 * D), lambda i: (i, 0)),
        out_shape=jax.ShapeDtypeStruct((n_blk * H, 2 * D), jnp.float32),
    )(tok_t)


def kernel(inputs, token_table, pos_table):
    B, S = inputs.shape
    V, D = token_table.shape
    idx = inputs.reshape(-1).astype(jnp.int32)
    tok2 = _convert_table(token_table.T)
    seq_per_w = B // _NUM_WORKERS

    mesh = plsc.VectorSubcoreMesh(core_axis_name="c", subcore_axis_name="s")
    NV = S // _L + (1 if S % _L else 0)

    @functools.partial(
        pl.kernel,
        mesh=mesh,
        out_type=jax.ShapeDtypeStruct((B, S, D), jnp.float32),
        scratch_types=[
            [pltpu.VMEM((S,), jnp.int32) for _ in range(2)],
            [pltpu.VMEM((S,), jnp.int32) for _ in range(2)],
            [pltpu.VMEM((S + 24,), jnp.int32) for _ in range(2)],
            [pltpu.VMEM((S, 2 * D), jnp.float32) for _ in range(2)],
            [pltpu.VMEM((S, D), jnp.float32) for _ in range(2)],
            pltpu.VMEM((S, D), jnp.float32),
            pltpu.SemaphoreType.DMA((2,)),
            pltpu.SemaphoreType.DMA((2,)),
            pltpu.SemaphoreType.DMA((2,)),
        ],
        compiler_params=pltpu.CompilerParams(needs_layout_passes=False),
    )
    def sc_kernel(idx_hbm, tok_hbm, pos_hbm, out_hbm, idxs, pidx, hv, gath,
                  outb, pos_v, gsem, ssem, isem):
        wid = lax.axis_index("s") * _NUM_CORES + lax.axis_index("c")
        base = wid * seq_per_w * S
        pltpu.sync_copy(pos_hbm, pos_v)

        def idx_start(s):
            b = s % 2
            return pltpu.async_copy(idx_hbm.at[pl.ds(base + s * S, S)],
                                    idxs[b], isem.at[b])

        def prep(s):
            b = s % 2
            for i in range(NV):
                o = min(i * _L, S - _L)
                v = idxs[b][pl.ds(o, _L)]
                pidx[b][pl.ds(o, _L)] = lax.bitwise_or(
                    lax.shift_left(lax.shift_right_logical(v, _SH), _SH - 1),
                    lax.bitwise_and(v, _HB - 1))
                hv[b][pl.ds(o, _L)] = lax.bitwise_and(
                    lax.shift_right_logical(v, _SH - 1), 1)

        def gather_start(s):
            b = s % 2
            return pltpu.async_copy(tok_hbm.at[pidx[b]], gath[b], gsem.at[b])

        def extract(s):
            b = s % 2

            def per_pair(q, _):
                h2 = hv[b][pl.ds(2 * q, _L)]
                for par in range(2):
                    r = 2 * q + par
                    m = jnp.full((_L,), h2[par], dtype=jnp.int32) != 0
                    for c in range(D // _L):
                        sl = pl.ds(c * _L, _L)
                        lo = gath[b][r, sl]
                        hi = gath[b][r, pl.ds(D + c * _L, _L)]
                        outb[b][r, sl] = jnp.where(m, hi, lo) + pos_v[r, sl]
                return ()

            lax.fori_loop(0, S // 2, per_pair, ())

        def store_start(s):
            return pltpu.async_copy(outb[s % 2],
                                    out_hbm.at[wid * seq_per_w + s],
                                    ssem.at[s % 2])

        st = [None] * seq_per_w
        iv = [idx_start(0), idx_start(1)]
        iv[0].wait()
        prep(0)
        gs = [gather_start(0), None]
        iv[0] = idx_start(2)
        iv[1].wait()
        prep(1)
        gs[1] = gather_start(1)
        iv[1] = idx_start(3)
        for s in range(seq_per_w):
            if s >= 2:
                st[s - 2].wait()
            gs[s % 2].wait()
            extract(s)
            if s + 2 < seq_per_w:
                iv[s % 2].wait()
                prep(s + 2)
                gs[s % 2] = gather_start(s + 2)
                if s + 4 < seq_per_w:
                    iv[s % 2] = idx_start(s + 4)
            st[s] = store_start(s)
        st[seq_per_w - 1].wait()
        st[seq_per_w - 2].wait()

    return sc_kernel(idx, tok2, pos_table)
